# baseline (speedup 1.0000x reference)
import functools

import jax
import jax.numpy as jnp
from jax import lax
from jax.experimental import pallas as pl
from jax.experimental.pallas import tpu as pltpu
from jax.experimental.pallas import tpu_sc as plsc

B = 32
S = 1024
D = 768
L = 16
NC = 2
NS = 16
NW = NC * NS
SCHUNK = S // NW
NBUF = 4
NH = B // L


def _body(ids_hbm, tok_hbm, pos_hbm, out_hbm,
          idx_t, oidx, pos_v, rows0, rows1, rows2, rows3,
          sg0, sg1, sg2, sg3, so0, so1, so2, so3):
    wid = lax.axis_index("s") * NC + lax.axis_index("c")
    s0 = pl.multiple_of(wid * SCHUNK, SCHUNK)

    bufs = (rows0, rows1, rows2, rows3)
    gsems = (sg0, sg1, sg2, sg3)
    osems = (so0, so1, so2, so3)

    ii = lax.iota(jnp.int32, L)
    for s in range(SCHUNK):
        for h in range(NH):
            oidx[s, pl.ds(h * L, L)] = (ii + h * L) * S + (s0 + s)
    for s in range(SCHUNK):
        for h in range(NH):
            pltpu.make_async_copy(
                ids_hbm.at[oidx.at[s, pl.ds(h * L, L)]],
                idx_t.at[pl.ds(s * B + h * L, L)], sg0).start()
    for s in range(SCHUNK):
        for h in range(NH):
            pltpu.make_async_copy(
                ids_hbm.at[oidx.at[s, pl.ds(h * L, L)]],
                idx_t.at[pl.ds(s * B + h * L, L)], sg0).wait()
    pltpu.sync_copy(pos_hbm.at[pl.ds(s0, SCHUNK)], pos_v)

    def gather_start(s, p):
        pltpu.make_async_copy(
            tok_hbm.at[idx_t.at[pl.ds(s * B, B)]], bufs[p], gsems[p]).start()

    def gather_wait(s, p):
        pltpu.make_async_copy(
            tok_hbm.at[idx_t.at[pl.ds(s * B, B)]], bufs[p], gsems[p]).wait()

    def out_start(s, p, h):
        pltpu.make_async_copy(
            bufs[p].at[pl.ds(h * L, L)],
            out_hbm.at[oidx.at[s, pl.ds(h * L, L)]], osems[p]).start()

    def out_wait(s, p, h):
        pltpu.make_async_copy(
            bufs[p].at[pl.ds(h * L, L)],
            out_hbm.at[oidx.at[s, pl.ds(h * L, L)]], osems[p]).wait()

    def add_pos(s, p, h):
        rows = bufs[p]

        @plsc.parallel_loop(0, D // L, 1)
        def add_col(j):
            c = pl.multiple_of(j * L, L)
            pj = pos_v[s, pl.ds(c, L)]
            for b in range(h * L, (h + 1) * L):
                plsc.addupdate(rows.at[b, pl.ds(c, L)], pj)

    def half(s, k):
        if isinstance(s, int):
            if s + 2 < SCHUNK:
                if s >= 2:
                    out_wait(s - 2, (s + 2) % NBUF, 0)
                    out_wait(s - 2, (s + 2) % NBUF, 1)
                gather_start(s + 2, (s + 2) % NBUF)
        else:
            out_wait(s - 2, (k + 2) % NBUF, 0)
            out_wait(s - 2, (k + 2) % NBUF, 1)
            gather_start(s + 2, (k + 2) % NBUF)
        gather_wait(s, k)
        add_pos(s, k, 0)
        out_start(s, k, 0)
        add_pos(s, k, 1)
        out_start(s, k, 1)

    gather_start(0, 0)
    gather_start(1, 1)
    for s in range(NBUF):
        half(s, s)

    def group(g, carry):
        j0 = g * NBUF
        for k in range(NBUF):
            half(j0 + k, k)
        return carry

    lax.fori_loop(1, SCHUNK // NBUF - 1, group, 0)

    for s in range(SCHUNK - NBUF, SCHUNK):
        half(s, s % NBUF)
    for s in range(SCHUNK - NBUF, SCHUNK):
        out_wait(s, s % NBUF, 0)
        out_wait(s, s % NBUF, 1)


@jax.jit
def kernel(input_ids, token_table, pos_table):
    mesh = plsc.VectorSubcoreMesh(core_axis_name="c", subcore_axis_name="s")
    f = functools.partial(
        pl.kernel,
        mesh=mesh,
        out_type=jax.ShapeDtypeStruct((B * S, D), jnp.float32),
        scratch_types=[
            pltpu.VMEM((SCHUNK * B,), jnp.int32),
            pltpu.VMEM((SCHUNK, B), jnp.int32),
            pltpu.VMEM((SCHUNK, D), jnp.float32),
            pltpu.VMEM((B, D), jnp.float32),
            pltpu.VMEM((B, D), jnp.float32),
            pltpu.VMEM((B, D), jnp.float32),
            pltpu.VMEM((B, D), jnp.float32),
            pltpu.SemaphoreType.DMA,
            pltpu.SemaphoreType.DMA,
            pltpu.SemaphoreType.DMA,
            pltpu.SemaphoreType.DMA,
            pltpu.SemaphoreType.DMA,
            pltpu.SemaphoreType.DMA,
            pltpu.SemaphoreType.DMA,
            pltpu.SemaphoreType.DMA,
        ],
    )(_body)
    out = f(input_ids.astype(jnp.int32).reshape(-1), token_table, pos_table)
    return out.reshape(B, S, D)


# speedup vs baseline: 1.0210x; 1.0210x over previous
import functools

import jax
import jax.numpy as jnp
from jax import lax
from jax.experimental import pallas as pl
from jax.experimental.pallas import tpu as pltpu
from jax.experimental.pallas import tpu_sc as plsc

B = 32
S = 1024
D = 768
L = 16
NC = 2
NS = 16
NW = NC * NS
SCHUNK = S // NW
NBUF = 4
NH = B // L


def _body(ids_hbm, tok_hbm, pos_hbm, out_hbm,
          idx_t, oidx, pos_v, rows0, rows1, rows2, rows3,
          sg0, sg1, sg2, sg3, so0, so1, so2, so3):
    wid = lax.axis_index("s") * NC + lax.axis_index("c")
    s0 = pl.multiple_of(wid * SCHUNK, SCHUNK)

    bufs = (rows0, rows1, rows2, rows3)
    gsems = (sg0, sg1, sg2, sg3)
    osems = (so0, so1, so2, so3)

    ii = lax.iota(jnp.int32, L)

    def stage_idx(s, sem):
        for h in range(NH):
            oidx[s, pl.ds(h * L, L)] = (ii + h * L) * S + (s0 + s)
        for h in range(NH):
            pltpu.make_async_copy(
                ids_hbm.at[oidx.at[s, pl.ds(h * L, L)]],
                idx_t.at[pl.ds(s * B + h * L, L)], sem).start()

    def stage_idx_wait(s, sem):
        for h in range(NH):
            pltpu.make_async_copy(
                ids_hbm.at[oidx.at[s, pl.ds(h * L, L)]],
                idx_t.at[pl.ds(s * B + h * L, L)], sem).wait()

    pos_cp = pltpu.make_async_copy(pos_hbm.at[pl.ds(s0, SCHUNK)], pos_v, so0)
    pos_cp.start()
    for s in range(2):
        stage_idx(s, so2)
        stage_idx_wait(s, so2)

    def gather_start(s, p):
        pltpu.make_async_copy(
            tok_hbm.at[idx_t.at[pl.ds(s * B, B)]], bufs[p], gsems[p]).start()

    def gather_wait(s, p):
        pltpu.make_async_copy(
            tok_hbm.at[idx_t.at[pl.ds(s * B, B)]], bufs[p], gsems[p]).wait()

    def out_start(s, p, h):
        pltpu.make_async_copy(
            bufs[p].at[pl.ds(h * L, L)],
            out_hbm.at[oidx.at[s, pl.ds(h * L, L)]], osems[p]).start()

    def out_wait(s, p, h):
        pltpu.make_async_copy(
            bufs[p].at[pl.ds(h * L, L)],
            out_hbm.at[oidx.at[s, pl.ds(h * L, L)]], osems[p]).wait()

    def add_pos(s, p, h):
        rows = bufs[p]

        @plsc.parallel_loop(0, D // L, 1)
        def add_col(j):
            c = pl.multiple_of(j * L, L)
            pj = pos_v[s, pl.ds(c, L)]
            for b in range(h * L, (h + 1) * L):
                plsc.addupdate(rows.at[b, pl.ds(c, L)], pj)

    def half(s, k):
        if isinstance(s, int):
            if s + 2 < SCHUNK:
                if s >= 2:
                    out_wait(s - 2, (s + 2) % NBUF, 0)
                    out_wait(s - 2, (s + 2) % NBUF, 1)
                gather_start(s + 2, (s + 2) % NBUF)
        else:
            out_wait(s - 2, (k + 2) % NBUF, 0)
            out_wait(s - 2, (k + 2) % NBUF, 1)
            gather_start(s + 2, (k + 2) % NBUF)
        gather_wait(s, k)
        add_pos(s, k, 0)
        out_start(s, k, 0)
        add_pos(s, k, 1)
        out_start(s, k, 1)

    gather_start(0, 0)
    gather_start(1, 1)
    for s in range(2, SCHUNK):
        stage_idx(s, so3)
    for s in range(2, SCHUNK):
        stage_idx_wait(s, so3)
    pos_cp.wait()
    for s in range(NBUF):
        half(s, s)

    def group(g, carry):
        j0 = g * NBUF
        for k in range(NBUF):
            half(j0 + k, k)
        return carry

    lax.fori_loop(1, SCHUNK // NBUF - 1, group, 0)

    for s in range(SCHUNK - NBUF, SCHUNK):
        half(s, s % NBUF)
    for s in range(SCHUNK - NBUF, SCHUNK):
        out_wait(s, s % NBUF, 0)
        out_wait(s, s % NBUF, 1)


@jax.jit
def kernel(input_ids, token_table, pos_table):
    mesh = plsc.VectorSubcoreMesh(core_axis_name="c", subcore_axis_name="s")
    f = functools.partial(
        pl.kernel,
        mesh=mesh,
        out_type=jax.ShapeDtypeStruct((B * S, D), jnp.float32),
        scratch_types=[
            pltpu.VMEM((SCHUNK * B,), jnp.int32),
            pltpu.VMEM((SCHUNK, B), jnp.int32),
            pltpu.VMEM((SCHUNK, D), jnp.float32),
            pltpu.VMEM((B, D), jnp.float32),
            pltpu.VMEM((B, D), jnp.float32),
            pltpu.VMEM((B, D), jnp.float32),
            pltpu.VMEM((B, D), jnp.float32),
            pltpu.SemaphoreType.DMA,
            pltpu.SemaphoreType.DMA,
            pltpu.SemaphoreType.DMA,
            pltpu.SemaphoreType.DMA,
            pltpu.SemaphoreType.DMA,
            pltpu.SemaphoreType.DMA,
            pltpu.SemaphoreType.DMA,
            pltpu.SemaphoreType.DMA,
        ],
    )(_body)
    out = f(input_ids.astype(jnp.int32).reshape(-1), token_table, pos_table)
    return out.reshape(B, S, D)
